# baseline (speedup 1.0000x reference)
import functools

import jax
import jax.numpy as jnp
from jax import lax
from jax.experimental import pallas as pl
from jax.experimental.pallas import tpu as pltpu
from jax.experimental.pallas import tpu_sc as plsc

_INTERPRET = False

N_NODES = 10000
N_EDGES = 160000
D = 128


def _mm_bias_body(x_ref, w_ref, b_ref, o_ref):
    o_ref[...] = (
        jnp.dot(x_ref[...], w_ref[...], preferred_element_type=jnp.float32)
        + b_ref[...]
    )


def _mm_bias(x, w, b, blk):
    n, k = x.shape
    m = w.shape[1]
    grid = n // blk
    return pl.pallas_call(
        _mm_bias_body,
        grid=(grid,),
        in_specs=[
            pl.BlockSpec((blk, k), lambda i: (i, 0)),
            pl.BlockSpec((k, m), lambda i: (0, 0)),
            pl.BlockSpec((1, m), lambda i: (0, 0)),
        ],
        out_specs=pl.BlockSpec((blk, m), lambda i: (i, 0)),
        out_shape=jax.ShapeDtypeStruct((n, m), jnp.float32),
        interpret=_INTERPRET,
    )(x, w, b.reshape(1, m))


def _embed_nodes_body(x_ref, w_ref, b_ref, ws_ref, wr_ref,
                      n_ref, ps_ref, pr_ref):
    n = (
        jnp.dot(x_ref[...], w_ref[...], preferred_element_type=jnp.float32)
        + b_ref[...]
    )
    n_ref[...] = n
    ps_ref[...] = jnp.dot(n, ws_ref[...], preferred_element_type=jnp.float32)
    pr_ref[...] = jnp.dot(n, wr_ref[...], preferred_element_type=jnp.float32)


def _embed_nodes(x, w, b, w1s, w1r, blk=2000):
    nn = x.shape[0]
    grid = nn // blk
    full = lambda i: (0, 0)
    rows = lambda i: (i, 0)
    return pl.pallas_call(
        _embed_nodes_body,
        grid=(grid,),
        in_specs=[
            pl.BlockSpec((blk, D), rows),
            pl.BlockSpec((D, D), full),
            pl.BlockSpec((1, D), full),
            pl.BlockSpec((D, D), full),
            pl.BlockSpec((D, D), full),
        ],
        out_specs=[pl.BlockSpec((blk, D), rows)] * 3,
        out_shape=[jax.ShapeDtypeStruct((nn, D), jnp.float32)] * 3,
        interpret=_INTERPRET,
    )(x, w, b.reshape(1, D), w1s, w1r)


def _make_edge_body(nagg):
    def _edge_body(x_ref, gs_ref, gr_ref, wa_ref, wb_ref, w2_ref, b2_ref,
                   g_ref, wg_ref, b1_ref, ev_ref, em_ref,
                   out_ref, agg_ref, acc_ref):
        a = jnp.dot(wa_ref[...], wb_ref[...],
                    preferred_element_type=jnp.float32)
        c = (
            jnp.dot(g_ref[...], wg_ref[...],
                    preferred_element_type=jnp.float32)
            + b1_ref[...]
            + jnp.dot(ev_ref[...], em_ref[...],
                      preferred_element_type=jnp.float32)
        )
        h = (
            jnp.dot(x_ref[...], a, preferred_element_type=jnp.float32)
            + gs_ref[...]
            + gr_ref[...]
            + c
        )
        y = (
            jnp.dot(jnp.maximum(h, 0.0), w2_ref[...],
                    preferred_element_type=jnp.float32)
            + b2_ref[...]
        )
        out_ref[...] = y
        i = pl.program_id(0)

        @pl.when(i == 0)
        def _():
            acc_ref[...] = jnp.zeros_like(acc_ref)

        @pl.when(i < nagg)
        def _():
            acc_ref[...] += jnp.sum(y, axis=0, keepdims=True)

        @pl.when(i == pl.num_programs(0) - 1)
        def _():
            agg_ref[...] = acc_ref[...]

    return _edge_body


def _edge_mlp(x, gs_rows, gr_rows, wa, wb, w2, b2, g, wg, b1, ev, em,
              ne=None, xoff=0, nreal=None, blk=1280):
    k = x.shape[1]
    if ne is None:
        ne = x.shape[0]
    if nreal is None:
        nreal = ne
    grid = ne // blk
    full = lambda i: (0, 0)
    rows = lambda i: (i, 0)
    xrows = lambda i: (i + xoff, 0)
    return pl.pallas_call(
        _make_edge_body(nreal // blk),
        grid=(grid,),
        in_specs=[
            pl.BlockSpec((blk, k), xrows),
            pl.BlockSpec((blk, D), rows),
            pl.BlockSpec((blk, D), rows),
            pl.BlockSpec((k, D), full),
            pl.BlockSpec((D, D), full),
            pl.BlockSpec((D, D), full),
            pl.BlockSpec((1, D), full),
            pl.BlockSpec((1, D), full),
            pl.BlockSpec((D, D), full),
            pl.BlockSpec((1, D), full),
            pl.BlockSpec((1, D), full),
            pl.BlockSpec((D, D), full),
        ],
        out_specs=[
            pl.BlockSpec((blk, D), rows),
            pl.BlockSpec((1, D), full),
        ],
        out_shape=[
            jax.ShapeDtypeStruct((ne, D), jnp.float32),
            jax.ShapeDtypeStruct((1, D), jnp.float32),
        ],
        scratch_shapes=[pltpu.VMEM((1, D), jnp.float32)],
        interpret=_INTERPRET,
    )(x, gs_rows, gr_rows, wa, wb, w2, b2.reshape(1, D),
      g, wg, b1.reshape(1, D), ev, em)


def _node_body_proj(n_ref, s1_ref, s2_ref, r1_ref, r2_ref,
                    vn_ref, vs_ref, vr_ref, g_ref,
                    vg_ref, b1_ref, v2_ref, b2_ref, ws_ref, wr_ref,
                    out_ref, agg_ref, ps_ref, pr_ref, acc_ref):
    c = (
        jnp.dot(g_ref[...], vg_ref[...], preferred_element_type=jnp.float32)
        + b1_ref[...]
    )
    h = (
        jnp.dot(n_ref[...], vn_ref[...], preferred_element_type=jnp.float32)
        + jnp.dot(s1_ref[...] + s2_ref[...], vs_ref[...],
                  preferred_element_type=jnp.float32)
        + jnp.dot(r1_ref[...] + r2_ref[...], vr_ref[...],
                  preferred_element_type=jnp.float32)
        + c
    )
    y = (
        jnp.dot(jnp.maximum(h, 0.0), v2_ref[...],
                preferred_element_type=jnp.float32)
        + b2_ref[...]
    )
    out_ref[...] = y
    if ps_ref is not None:
        ps_ref[...] = jnp.dot(y, ws_ref[...],
                              preferred_element_type=jnp.float32)
        pr_ref[...] = jnp.dot(y, wr_ref[...],
                              preferred_element_type=jnp.float32)
    i = pl.program_id(0)

    @pl.when(i == 0)
    def _():
        acc_ref[...] = jnp.zeros_like(acc_ref)

    acc_ref[...] += jnp.sum(y, axis=0, keepdims=True)

    @pl.when(i == pl.num_programs(0) - 1)
    def _():
        agg_ref[...] = acc_ref[...]


def _node_body_noproj(n_ref, s1_ref, s2_ref, r1_ref, r2_ref,
                      vn_ref, vs_ref, vr_ref, g_ref,
                      vg_ref, b1_ref, v2_ref, b2_ref,
                      out_ref, agg_ref, acc_ref):
    _node_body_proj(n_ref, s1_ref, s2_ref, r1_ref, r2_ref,
                    vn_ref, vs_ref, vr_ref, g_ref,
                    vg_ref, b1_ref, v2_ref, b2_ref, None, None,
                    out_ref, agg_ref, None, None, acc_ref)


def _node_mlp(n, s1, s2, rv1, rv2, vn, vs, vr, g, vg, b1, v2, b2,
              ws_next=None, wr_next=None, blk=2000):
    nn = n.shape[0]
    grid = nn // blk
    full = lambda i: (0, 0)
    rows = lambda i: (i, 0)
    with_proj = ws_next is not None
    in_specs = [
        pl.BlockSpec((blk, D), rows),
        pl.BlockSpec((blk, D), rows),
        pl.BlockSpec((blk, D), rows),
        pl.BlockSpec((blk, D), rows),
        pl.BlockSpec((blk, D), rows),
        pl.BlockSpec((D, D), full),
        pl.BlockSpec((D, D), full),
        pl.BlockSpec((D, D), full),
        pl.BlockSpec((1, D), full),
        pl.BlockSpec((D, D), full),
        pl.BlockSpec((1, D), full),
        pl.BlockSpec((D, D), full),
        pl.BlockSpec((1, D), full),
    ]
    out_specs = [pl.BlockSpec((blk, D), rows), pl.BlockSpec((1, D), full)]
    out_shape = [
        jax.ShapeDtypeStruct((nn, D), jnp.float32),
        jax.ShapeDtypeStruct((1, D), jnp.float32),
    ]
    args = [n, s1, s2, rv1, rv2, vn, vs, vr, g, vg, b1.reshape(1, D), v2,
            b2.reshape(1, D)]
    if with_proj:
        in_specs += [pl.BlockSpec((D, D), full)] * 2
        out_specs += [pl.BlockSpec((blk, D), rows)] * 2
        out_shape += [jax.ShapeDtypeStruct((nn, D), jnp.float32)] * 2
        args += [ws_next, wr_next]
    return pl.pallas_call(
        _node_body_proj if with_proj else _node_body_noproj,
        grid=(grid,),
        in_specs=in_specs,
        out_specs=out_specs,
        out_shape=out_shape,
        scratch_shapes=[pltpu.VMEM((1, D), jnp.float32)],
        interpret=_INTERPRET,
    )(*args)


def _glob_body(na_ref, ea1_ref, ea2_ref, g_ref, un_ref, ue_ref, ug_ref,
               b1_ref, u2_ref, b2_ref, o_ref):
    ea = ea1_ref[...] + ea2_ref[...]
    h = (
        jnp.dot(na_ref[...], un_ref[...], preferred_element_type=jnp.float32)
        + jnp.dot(ea, ue_ref[...], preferred_element_type=jnp.float32)
        + jnp.dot(g_ref[...], ug_ref[...], preferred_element_type=jnp.float32)
        + b1_ref[...]
    )
    o_ref[...] = (
        jnp.dot(jnp.maximum(h, 0.0), u2_ref[...],
                preferred_element_type=jnp.float32)
        + b2_ref[...]
    )


def _glob_mlp(na, ea1, ea2, g, un, ue, ug, b1, u2, b2):
    return pl.pallas_call(
        _glob_body,
        in_specs=[pl.BlockSpec(s, None) for s in
                  [(1, D), (1, D), (1, D), (1, D), (D, D), (D, D), (D, D),
                   (1, D), (D, D), (1, D)]],
        out_specs=pl.BlockSpec((1, D), None),
        out_shape=jax.ShapeDtypeStruct((1, D), jnp.float32),
        interpret=_INTERPRET,
    )(na, ea1, ea2, g, un, ue, ug, b1.reshape(1, D), u2, b2.reshape(1, D))


def _readout_body(g_ref, w1_ref, b1_ref, w2t_ref, b2_ref, o_ref):
    h = (
        jnp.dot(g_ref[...], w1_ref[...], preferred_element_type=jnp.float32)
        + b1_ref[...]
    )
    h = jnp.maximum(h, 0.0)
    o_ref[...] = (
        jnp.sum(h * w2t_ref[...], axis=1, keepdims=True) + b2_ref[...]
    )


def _readout(g, w1, b1, w2, b2):
    return pl.pallas_call(
        _readout_body,
        in_specs=[pl.BlockSpec(s, None) for s in
                  [(1, D), (D, D), (1, D), (1, D), (1, 1)]],
        out_specs=pl.BlockSpec((1, 1), None),
        out_shape=jax.ShapeDtypeStruct((1, 1), jnp.float32),
        interpret=_INTERPRET,
    )(g, w1, b1.reshape(1, D), w2.reshape(1, D), b2.reshape(1, 1))


_CH = 128
_H1C = 40
_H2C = 40
E_PAD = 163840
N_PAD = 10240
_NSL = N_PAD // 16
_ZCH = 32
_SC_MESH = dict(core_axis_name="c", subcore_axis_name="s",
                num_cores=2, num_subcores=16)


_NBUF = 4


def _gather_one(tab_hbm, idx3d, out_hbm, idx_v, rows, gsems, wsems, ss, cpw):
    nrnd = cpw // _NBUF
    tail = cpw - nrnd * _NBUF
    ebase = ss * cpw * _CH
    pltpu.sync_copy(idx3d.at[ss], idx_v)

    for b in range(_NBUF):
        pltpu.async_copy(tab_hbm.at[idx_v.at[b]], rows[b], gsems[b])

    def rnd(r):
        for b in range(_NBUF):
            k = r * _NBUF + b
            pltpu.make_async_copy(tab_hbm.at[idx_v.at[b]], rows[b],
                                  gsems[b]).wait()
            pltpu.async_copy(rows[b],
                             out_hbm.at[pl.ds(ebase + k * _CH, _CH)],
                             wsems[b])

        @pl.when(r < nrnd - 1)
        def _():
            for b in range(_NBUF):
                pltpu.make_async_copy(rows[b],
                                      out_hbm.at[pl.ds(ebase, _CH)],
                                      wsems[b]).wait()
                pltpu.async_copy(tab_hbm.at[idx_v.at[(r + 1) * _NBUF + b]],
                                 rows[b], gsems[b])

        @pl.when(r == nrnd - 1)
        def _():
            for b in range(_NBUF):
                pltpu.make_async_copy(rows[b],
                                      out_hbm.at[pl.ds(ebase, _CH)],
                                      wsems[b]).wait()

    pl.loop(0, nrnd)(rnd)

    for t in range(tail):
        k = nrnd * _NBUF + t
        pltpu.async_copy(tab_hbm.at[idx_v.at[k]], rows[t], gsems[t])
    for t in range(tail):
        k = nrnd * _NBUF + t
        pltpu.make_async_copy(tab_hbm.at[idx_v.at[k]], rows[t],
                              gsems[t]).wait()
        pltpu.async_copy(rows[t], out_hbm.at[pl.ds(ebase + k * _CH, _CH)],
                         wsems[t])
    for t in range(tail):
        pltpu.make_async_copy(rows[t], out_hbm.at[pl.ds(ebase, _CH)],
                              wsems[t]).wait()


def _sc_gather(ps, pr, s3d, r3d, cpw):
    ne = 16 * cpw * _CH

    def body(ps_hbm, pr_hbm, s3_hbm, r3_hbm, gs_hbm, gr_hbm, idx_v, *bufs):
        c = lax.axis_index("c")
        ss = lax.axis_index("s")
        rows = list(bufs[:_NBUF])
        gsems = list(bufs[_NBUF:2 * _NBUF])
        wsems = list(bufs[2 * _NBUF:])

        @pl.when(c == 0)
        def _():
            _gather_one(ps_hbm, s3_hbm, gs_hbm, idx_v, rows, gsems, wsems,
                        ss, cpw)

        @pl.when(c == 1)
        def _():
            _gather_one(pr_hbm, r3_hbm, gr_hbm, idx_v, rows, gsems, wsems,
                        ss, cpw)

    f = pl.kernel(
        body,
        out_type=[
            jax.ShapeDtypeStruct((ne, D), jnp.float32),
            jax.ShapeDtypeStruct((ne, D), jnp.float32),
        ],
        mesh=plsc.VectorSubcoreMesh(**_SC_MESH),
        scratch_types=(
            [pltpu.VMEM((cpw, _CH), jnp.int32)]
            + [pltpu.VMEM((_CH, D), jnp.float32) for _ in range(_NBUF)]
            + [pltpu.SemaphoreType.DMA for _ in range(2 * _NBUF)]
        ),
    )
    return f(ps, pr, s3d, r3d)


_SNB = 2


def _sc_segment_sums(e, s3d, r3d, cpw):
    snr = cpw // _SNB
    tail = cpw - snr * _SNB

    def body(e_hbm, s3_hbm, r3_hbm, sent_hbm, recv_hbm,
             acc, idx_v, *bufs):
        c = lax.axis_index("c")
        ss = lax.axis_index("s")
        slice_base = ss * _NSL
        rows = list(bufs[:_SNB])
        rsems = list(bufs[_SNB:2 * _SNB])
        ssems = list(bufs[2 * _SNB:])

        def zrow(i):
            for j in range(8):
                rows[0][i, pl.ds(j * 16, 16)] = jnp.zeros((16,), jnp.float32)

        pl.loop(0, _CH)(zrow)

        def zcp(i):
            pltpu.sync_copy(rows[0],
                            acc.at[pl.ds(slice_base + i * _CH, _CH)])

        pl.loop(0, _NSL // _CH)(zcp)

        @pl.when(c == 0)
        def _():
            pltpu.sync_copy(s3_hbm.at[ss], idx_v)

        @pl.when(c == 1)
        def _():
            pltpu.sync_copy(r3_hbm.at[ss], idx_v)

        plsc.subcore_barrier()

        ebase = ss * cpw * _CH

        for b in range(_SNB):
            pltpu.async_copy(e_hbm.at[pl.ds(ebase + b * _CH, _CH)], rows[b],
                             rsems[b])

        def rnd(r):
            for b in range(_SNB):
                pltpu.make_async_copy(e_hbm.at[pl.ds(ebase, _CH)], rows[b],
                                      rsems[b]).wait()
                pltpu.async_copy(rows[b], acc.at[idx_v.at[r * _SNB + b]],
                                 ssems[b], add=True)

            @pl.when(r < snr - 1)
            def _():
                for b in range(_SNB):
                    pltpu.make_async_copy(rows[b], acc.at[idx_v.at[b]],
                                          ssems[b]).wait()
                    k = (r + 1) * _SNB + b
                    pltpu.async_copy(e_hbm.at[pl.ds(ebase + k * _CH, _CH)],
                                     rows[b], rsems[b])

            @pl.when(r == snr - 1)
            def _():
                for b in range(_SNB):
                    pltpu.make_async_copy(rows[b], acc.at[idx_v.at[b]],
                                          ssems[b]).wait()

        pl.loop(0, snr)(rnd)

        for t in range(tail):
            k = snr * _SNB + t
            pltpu.async_copy(e_hbm.at[pl.ds(ebase + k * _CH, _CH)],
                             rows[t], rsems[t])
        for t in range(tail):
            k = snr * _SNB + t
            pltpu.make_async_copy(e_hbm.at[pl.ds(ebase, _CH)], rows[t],
                                  rsems[t]).wait()
            pltpu.async_copy(rows[t], acc.at[idx_v.at[k]], ssems[t],
                             add=True)
        for t in range(tail):
            pltpu.make_async_copy(rows[t], acc.at[idx_v.at[0]],
                                  ssems[t]).wait()

        plsc.subcore_barrier()

        def wcp(i):
            sl = pl.ds(slice_base + i * _CH, _CH)

            @pl.when(c == 0)
            def _():
                pltpu.sync_copy(acc.at[sl], sent_hbm.at[sl])

            @pl.when(c == 1)
            def _():
                pltpu.sync_copy(acc.at[sl], recv_hbm.at[sl])

        pl.loop(0, _NSL // _CH)(wcp)

    f = pl.kernel(
        body,
        out_type=[
            jax.ShapeDtypeStruct((N_PAD, D), jnp.float32),
            jax.ShapeDtypeStruct((N_PAD, D), jnp.float32),
        ],
        mesh=plsc.VectorSubcoreMesh(**_SC_MESH),
        scratch_types=(
            [
                pltpu.VMEM_SHARED((N_PAD, D), jnp.float32),
                pltpu.VMEM((cpw, _CH), jnp.int32),
            ]
            + [pltpu.VMEM((_CH, D), jnp.float32) for _ in range(_SNB)]
            + [pltpu.SemaphoreType.DMA for _ in range(2 * _SNB)]
        ),
    )
    return f(e, s3d, r3d)


def _softplus(x):
    return jnp.log(1.0 + jnp.exp(x))


def _sample_mlp(layers, key):
    ks = jax.random.split(key, len(layers))
    out = []
    for p, k in zip(layers, ks):
        w = p['w_mu'] + jax.random.normal(k, p['w_mu'].shape,
                                          dtype=jnp.float32) * _softplus(p['w_rho'])
        b = p['b_mu'] + jax.random.normal(k, p['b_mu'].shape,
                                          dtype=jnp.float32) * _softplus(p['b_rho'])
        out.append((w, b))
    return out


def kernel(nodes, edges, senders, receivers, globals_, positions, box, params):
    keys = jax.random.split(jax.random.key(42), 4)
    emb = params['embed']

    zero_vec = jnp.zeros((1, D), jnp.float32)
    eye = jnp.eye(D, dtype=jnp.float32)
    h1 = 16 * _H1C * _CH
    npad = E_PAD - N_EDGES
    gpad = jnp.zeros((npad,), jnp.int32)
    spad = jnp.full((npad,), N_NODES, jnp.int32)
    s3d1 = senders[:h1].reshape(16, _H1C, _CH)
    r3d1 = receivers[:h1].reshape(16, _H1C, _CH)
    sg2 = jnp.concatenate([senders[h1:], gpad]).reshape(16, _H2C, _CH)
    rg2 = jnp.concatenate([receivers[h1:], gpad]).reshape(16, _H2C, _CH)
    ss2 = jnp.concatenate([senders[h1:], spad]).reshape(16, _H2C, _CH)
    rs2 = jnp.concatenate([receivers[h1:], spad]).reshape(16, _H2C, _CH)
    edges_pad = jnp.concatenate(
        [edges, jnp.zeros((npad, edges.shape[1]), jnp.float32)])

    sw = []
    for s in range(2):
        sp = params['steps'][s]
        k_e, k_n, k_g = jax.random.split(keys[s], 3)
        sw.append((_sample_mlp(sp['edge'], k_e),
                   _sample_mlp(sp['node'], k_n),
                   _sample_mlp(sp['glob'], k_g)))
    (rw1, rb1), (rw2, rb2) = _sample_mlp(params['readout'], keys[-1])

    ew1_0 = sw[0][0][0][0]
    n, ps, pr = _embed_nodes(nodes, emb['node_w'], emb['node_b'],
                             ew1_0[D:2 * D], ew1_0[2 * D:3 * D])
    g = _mm_bias(globals_, emb['glob_w'], emb['glob_b'], blk=1)

    e1 = e2 = None
    for s in range(2):
        (ew1, eb1), (ew2, eb2) = sw[s][0]
        (nw1, nb1), (nw2, nb2) = sw[s][1]
        (gw1, gb1), (gw2, gb2) = sw[s][2]
        w1e = ew1[0:D]
        w1g = ew1[3 * D:4 * D]

        if s == 0:
            x1, x2, xoff2 = edges_pad, edges_pad, h1 // 1280
            wa, wb = emb['edge_w'], w1e
            ev, em = emb['edge_b'].reshape(1, D), w1e
        else:
            x1, x2, xoff2 = e1, e2, 0
            wa, wb = w1e, eye
            ev, em = zero_vec, eye

        gs1, gr1 = _sc_gather(ps, pr, s3d1, r3d1, _H1C)
        gs2, gr2 = _sc_gather(ps, pr, sg2, rg2, _H2C)
        e1, ea1 = _edge_mlp(x1, gs1, gr1, wa, wb, ew2, eb2,
                            g, w1g, eb1, ev, em, ne=h1, xoff=0)
        sp1, rp1 = _sc_segment_sums(e1, s3d1, r3d1, _H1C)
        e2, ea2 = _edge_mlp(x2, gs2, gr2, wa, wb, ew2, eb2,
                            g, w1g, eb1, ev, em,
                            ne=E_PAD - h1, xoff=xoff2,
                            nreal=N_EDGES - h1)
        sp2, rp2 = _sc_segment_sums(e2, ss2, rs2, _H2C)

        if s == 0:
            ew1_n = sw[1][0][0][0]
            n, n_agg, ps, pr = _node_mlp(
                n, sp1, sp2, rp1, rp2,
                nw1[0:D], nw1[D:2 * D], nw1[2 * D:3 * D],
                g, nw1[3 * D:4 * D], nb1, nw2, nb2,
                ws_next=ew1_n[D:2 * D], wr_next=ew1_n[2 * D:3 * D])
        else:
            n, n_agg = _node_mlp(
                n, sp1, sp2, rp1, rp2,
                nw1[0:D], nw1[D:2 * D], nw1[2 * D:3 * D],
                g, nw1[3 * D:4 * D], nb1, nw2, nb2)
        g = _glob_mlp(n_agg, ea1, ea2, g,
                      gw1[0:D], gw1[D:2 * D], gw1[2 * D:3 * D], gb1,
                      gw2, gb2)

    return _readout(g, rw1, rb1, rw2, rb2)


# speedup vs baseline: 1.4720x; 1.4720x over previous
import functools

import jax
import jax.numpy as jnp
from jax import lax
from jax.experimental import pallas as pl
from jax.experimental.pallas import tpu as pltpu
from jax.experimental.pallas import tpu_sc as plsc

_INTERPRET = False

N_NODES = 10000
N_EDGES = 160000
D = 128


def _mm_bias_body(x_ref, w_ref, b_ref, o_ref):
    o_ref[...] = (
        jnp.dot(x_ref[...], w_ref[...], preferred_element_type=jnp.float32)
        + b_ref[...]
    )


def _mm_bias(x, w, b, blk):
    n, k = x.shape
    m = w.shape[1]
    grid = n // blk
    return pl.pallas_call(
        _mm_bias_body,
        grid=(grid,),
        in_specs=[
            pl.BlockSpec((blk, k), lambda i: (i, 0)),
            pl.BlockSpec((k, m), lambda i: (0, 0)),
            pl.BlockSpec((1, m), lambda i: (0, 0)),
        ],
        out_specs=pl.BlockSpec((blk, m), lambda i: (i, 0)),
        out_shape=jax.ShapeDtypeStruct((n, m), jnp.float32),
        interpret=_INTERPRET,
    )(x, w, b.reshape(1, m))


def _embed_nodes_body(x_ref, w_ref, b_ref, ws_ref, wr_ref,
                      n_ref, ps_ref, pr_ref):
    n = (
        jnp.dot(x_ref[...], w_ref[...], preferred_element_type=jnp.float32)
        + b_ref[...]
    )
    n_ref[...] = n
    ps_ref[...] = jnp.dot(n, ws_ref[...], preferred_element_type=jnp.float32)
    pr_ref[...] = jnp.dot(n, wr_ref[...], preferred_element_type=jnp.float32)


def _embed_nodes(x, w, b, w1s, w1r, blk=2000):
    nn = x.shape[0]
    grid = nn // blk
    full = lambda i: (0, 0)
    rows = lambda i: (i, 0)
    return pl.pallas_call(
        _embed_nodes_body,
        grid=(grid,),
        in_specs=[
            pl.BlockSpec((blk, D), rows),
            pl.BlockSpec((D, D), full),
            pl.BlockSpec((1, D), full),
            pl.BlockSpec((D, D), full),
            pl.BlockSpec((D, D), full),
        ],
        out_specs=[pl.BlockSpec((blk, D), rows)] * 3,
        out_shape=[jax.ShapeDtypeStruct((nn, D), jnp.float32)] * 3,
        interpret=_INTERPRET,
    )(x, w, b.reshape(1, D), w1s, w1r)


def _edge_body(x_ref, gs_ref, gr_ref, wa_ref, wb_ref, w2_ref, b2_ref,
               g_ref, wg_ref, b1_ref, ev_ref, em_ref,
               out_ref, agg_ref, acc_ref):
    a = jnp.dot(wa_ref[...], wb_ref[...], preferred_element_type=jnp.float32)
    c = (
        jnp.dot(g_ref[...], wg_ref[...], preferred_element_type=jnp.float32)
        + b1_ref[...]
        + jnp.dot(ev_ref[...], em_ref[...], preferred_element_type=jnp.float32)
    )
    h = (
        jnp.dot(x_ref[...], a, preferred_element_type=jnp.float32)
        + gs_ref[...]
        + gr_ref[...]
        + c
    )
    y = (
        jnp.dot(jnp.maximum(h, 0.0), w2_ref[...],
                preferred_element_type=jnp.float32)
        + b2_ref[...]
    )
    out_ref[...] = y
    i = pl.program_id(0)

    @pl.when(i == 0)
    def _():
        acc_ref[...] = jnp.zeros_like(acc_ref)

    acc_ref[...] += jnp.sum(y, axis=0, keepdims=True)

    @pl.when(i == pl.num_programs(0) - 1)
    def _():
        agg_ref[...] = acc_ref[...]


def _edge_mlp(x, gs_rows, gr_rows, wa, wb, w2, b2, g, wg, b1, ev, em,
              ne=None, xoff=0, blk=1280):
    k = x.shape[1]
    if ne is None:
        ne = x.shape[0]
    grid = ne // blk
    full = lambda i: (0, 0)
    rows = lambda i: (i, 0)
    xrows = lambda i: (i + xoff, 0)
    return pl.pallas_call(
        _edge_body,
        grid=(grid,),
        in_specs=[
            pl.BlockSpec((blk, k), xrows),
            pl.BlockSpec((blk, D), rows),
            pl.BlockSpec((blk, D), rows),
            pl.BlockSpec((k, D), full),
            pl.BlockSpec((D, D), full),
            pl.BlockSpec((D, D), full),
            pl.BlockSpec((1, D), full),
            pl.BlockSpec((1, D), full),
            pl.BlockSpec((D, D), full),
            pl.BlockSpec((1, D), full),
            pl.BlockSpec((1, D), full),
            pl.BlockSpec((D, D), full),
        ],
        out_specs=[
            pl.BlockSpec((blk, D), rows),
            pl.BlockSpec((1, D), full),
        ],
        out_shape=[
            jax.ShapeDtypeStruct((ne, D), jnp.float32),
            jax.ShapeDtypeStruct((1, D), jnp.float32),
        ],
        scratch_shapes=[pltpu.VMEM((1, D), jnp.float32)],
        interpret=_INTERPRET,
    )(x, gs_rows, gr_rows, wa, wb, w2, b2.reshape(1, D),
      g, wg, b1.reshape(1, D), ev, em)


def _node_body_proj(n_ref, s1_ref, s2_ref, r1_ref, r2_ref,
                    vn_ref, vs_ref, vr_ref, g_ref,
                    vg_ref, b1_ref, v2_ref, b2_ref, ws_ref, wr_ref,
                    out_ref, agg_ref, ps_ref, pr_ref, acc_ref):
    c = (
        jnp.dot(g_ref[...], vg_ref[...], preferred_element_type=jnp.float32)
        + b1_ref[...]
    )
    h = (
        jnp.dot(n_ref[...], vn_ref[...], preferred_element_type=jnp.float32)
        + jnp.dot(s1_ref[...] + s2_ref[...], vs_ref[...],
                  preferred_element_type=jnp.float32)
        + jnp.dot(r1_ref[...] + r2_ref[...], vr_ref[...],
                  preferred_element_type=jnp.float32)
        + c
    )
    y = (
        jnp.dot(jnp.maximum(h, 0.0), v2_ref[...],
                preferred_element_type=jnp.float32)
        + b2_ref[...]
    )
    out_ref[...] = y
    if ps_ref is not None:
        ps_ref[...] = jnp.dot(y, ws_ref[...],
                              preferred_element_type=jnp.float32)
        pr_ref[...] = jnp.dot(y, wr_ref[...],
                              preferred_element_type=jnp.float32)
    i = pl.program_id(0)

    @pl.when(i == 0)
    def _():
        acc_ref[...] = jnp.zeros_like(acc_ref)

    acc_ref[...] += jnp.sum(y, axis=0, keepdims=True)

    @pl.when(i == pl.num_programs(0) - 1)
    def _():
        agg_ref[...] = acc_ref[...]


def _node_body_noproj(n_ref, s1_ref, s2_ref, r1_ref, r2_ref,
                      vn_ref, vs_ref, vr_ref, g_ref,
                      vg_ref, b1_ref, v2_ref, b2_ref,
                      out_ref, agg_ref, acc_ref):
    _node_body_proj(n_ref, s1_ref, s2_ref, r1_ref, r2_ref,
                    vn_ref, vs_ref, vr_ref, g_ref,
                    vg_ref, b1_ref, v2_ref, b2_ref, None, None,
                    out_ref, agg_ref, None, None, acc_ref)


def _node_mlp(n, s1, s2, rv1, rv2, vn, vs, vr, g, vg, b1, v2, b2,
              ws_next=None, wr_next=None, blk=2000):
    nn = n.shape[0]
    grid = nn // blk
    full = lambda i: (0, 0)
    rows = lambda i: (i, 0)
    with_proj = ws_next is not None
    in_specs = [
        pl.BlockSpec((blk, D), rows),
        pl.BlockSpec((blk, D), rows),
        pl.BlockSpec((blk, D), rows),
        pl.BlockSpec((blk, D), rows),
        pl.BlockSpec((blk, D), rows),
        pl.BlockSpec((D, D), full),
        pl.BlockSpec((D, D), full),
        pl.BlockSpec((D, D), full),
        pl.BlockSpec((1, D), full),
        pl.BlockSpec((D, D), full),
        pl.BlockSpec((1, D), full),
        pl.BlockSpec((D, D), full),
        pl.BlockSpec((1, D), full),
    ]
    out_specs = [pl.BlockSpec((blk, D), rows), pl.BlockSpec((1, D), full)]
    out_shape = [
        jax.ShapeDtypeStruct((nn, D), jnp.float32),
        jax.ShapeDtypeStruct((1, D), jnp.float32),
    ]
    args = [n, s1, s2, rv1, rv2, vn, vs, vr, g, vg, b1.reshape(1, D), v2,
            b2.reshape(1, D)]
    if with_proj:
        in_specs += [pl.BlockSpec((D, D), full)] * 2
        out_specs += [pl.BlockSpec((blk, D), rows)] * 2
        out_shape += [jax.ShapeDtypeStruct((nn, D), jnp.float32)] * 2
        args += [ws_next, wr_next]
    return pl.pallas_call(
        _node_body_proj if with_proj else _node_body_noproj,
        grid=(grid,),
        in_specs=in_specs,
        out_specs=out_specs,
        out_shape=out_shape,
        scratch_shapes=[pltpu.VMEM((1, D), jnp.float32)],
        interpret=_INTERPRET,
    )(*args)


def _glob_body(na_ref, ea1_ref, ea2_ref, g_ref, un_ref, ue_ref, ug_ref,
               b1_ref, u2_ref, b2_ref, o_ref):
    ea = ea1_ref[...] + ea2_ref[...]
    h = (
        jnp.dot(na_ref[...], un_ref[...], preferred_element_type=jnp.float32)
        + jnp.dot(ea, ue_ref[...], preferred_element_type=jnp.float32)
        + jnp.dot(g_ref[...], ug_ref[...], preferred_element_type=jnp.float32)
        + b1_ref[...]
    )
    o_ref[...] = (
        jnp.dot(jnp.maximum(h, 0.0), u2_ref[...],
                preferred_element_type=jnp.float32)
        + b2_ref[...]
    )


def _glob_mlp(na, ea1, ea2, g, un, ue, ug, b1, u2, b2):
    return pl.pallas_call(
        _glob_body,
        in_specs=[pl.BlockSpec(s, None) for s in
                  [(1, D), (1, D), (1, D), (1, D), (D, D), (D, D), (D, D),
                   (1, D), (D, D), (1, D)]],
        out_specs=pl.BlockSpec((1, D), None),
        out_shape=jax.ShapeDtypeStruct((1, D), jnp.float32),
        interpret=_INTERPRET,
    )(na, ea1, ea2, g, un, ue, ug, b1.reshape(1, D), u2, b2.reshape(1, D))


def _readout_body(g_ref, w1_ref, b1_ref, w2t_ref, b2_ref, o_ref):
    h = (
        jnp.dot(g_ref[...], w1_ref[...], preferred_element_type=jnp.float32)
        + b1_ref[...]
    )
    h = jnp.maximum(h, 0.0)
    o_ref[...] = (
        jnp.sum(h * w2t_ref[...], axis=1, keepdims=True) + b2_ref[...]
    )


def _readout(g, w1, b1, w2, b2):
    return pl.pallas_call(
        _readout_body,
        in_specs=[pl.BlockSpec(s, None) for s in
                  [(1, D), (D, D), (1, D), (1, D), (1, 1)]],
        out_specs=pl.BlockSpec((1, 1), None),
        out_shape=jax.ShapeDtypeStruct((1, 1), jnp.float32),
        interpret=_INTERPRET,
    )(g, w1, b1.reshape(1, D), w2.reshape(1, D), b2.reshape(1, 1))


_CH = 80
_H1C = 64
_H2C = 61
N_PAD = 10240
_NSL = N_PAD // 16
_ZCH = 32
_SC_MESH = dict(core_axis_name="c", subcore_axis_name="s",
                num_cores=2, num_subcores=16)


_NBUF = 5


def _gather_one(tab_hbm, idx3d, out_hbm, idx_v, rows, gsems, wsems, ss, cpw):
    nrnd = cpw // _NBUF
    tail = cpw - nrnd * _NBUF
    ebase = ss * cpw * _CH
    pltpu.sync_copy(idx3d.at[ss], idx_v)

    for b in range(_NBUF):
        pltpu.async_copy(tab_hbm.at[idx_v.at[b]], rows[b], gsems[b])

    def rnd(r):
        for b in range(_NBUF):
            k = r * _NBUF + b
            pltpu.make_async_copy(tab_hbm.at[idx_v.at[b]], rows[b],
                                  gsems[b]).wait()
            pltpu.async_copy(rows[b],
                             out_hbm.at[pl.ds(ebase + k * _CH, _CH)],
                             wsems[b])

        @pl.when(r < nrnd - 1)
        def _():
            for b in range(_NBUF):
                pltpu.make_async_copy(rows[b],
                                      out_hbm.at[pl.ds(ebase, _CH)],
                                      wsems[b]).wait()
                pltpu.async_copy(tab_hbm.at[idx_v.at[(r + 1) * _NBUF + b]],
                                 rows[b], gsems[b])

        @pl.when(r == nrnd - 1)
        def _():
            for b in range(_NBUF):
                pltpu.make_async_copy(rows[b],
                                      out_hbm.at[pl.ds(ebase, _CH)],
                                      wsems[b]).wait()

    pl.loop(0, nrnd)(rnd)

    for t in range(tail):
        k = nrnd * _NBUF + t
        pltpu.async_copy(tab_hbm.at[idx_v.at[k]], rows[t], gsems[t])
    for t in range(tail):
        k = nrnd * _NBUF + t
        pltpu.make_async_copy(tab_hbm.at[idx_v.at[k]], rows[t],
                              gsems[t]).wait()
        pltpu.async_copy(rows[t], out_hbm.at[pl.ds(ebase + k * _CH, _CH)],
                         wsems[t])
    for t in range(tail):
        pltpu.make_async_copy(rows[t], out_hbm.at[pl.ds(ebase, _CH)],
                              wsems[t]).wait()


def _sc_gather(ps, pr, s3d, r3d, cpw):
    ne = 16 * cpw * _CH

    def body(ps_hbm, pr_hbm, s3_hbm, r3_hbm, gs_hbm, gr_hbm, idx_v, *bufs):
        c = lax.axis_index("c")
        ss = lax.axis_index("s")
        rows = list(bufs[:_NBUF])
        gsems = list(bufs[_NBUF:2 * _NBUF])
        wsems = list(bufs[2 * _NBUF:])

        @pl.when(c == 0)
        def _():
            _gather_one(ps_hbm, s3_hbm, gs_hbm, idx_v, rows, gsems, wsems,
                        ss, cpw)

        @pl.when(c == 1)
        def _():
            _gather_one(pr_hbm, r3_hbm, gr_hbm, idx_v, rows, gsems, wsems,
                        ss, cpw)

    f = pl.kernel(
        body,
        out_type=[
            jax.ShapeDtypeStruct((ne, D), jnp.float32),
            jax.ShapeDtypeStruct((ne, D), jnp.float32),
        ],
        mesh=plsc.VectorSubcoreMesh(**_SC_MESH),
        scratch_types=(
            [pltpu.VMEM((cpw, _CH), jnp.int32)]
            + [pltpu.VMEM((_CH, D), jnp.float32) for _ in range(_NBUF)]
            + [pltpu.SemaphoreType.DMA for _ in range(2 * _NBUF)]
        ),
    )
    return f(ps, pr, s3d, r3d)


_SNB = 3


def _sc_segment_sums(e, s3d, r3d, cpw):
    snr = cpw // _SNB
    tail = cpw - snr * _SNB

    def body(e_hbm, s3_hbm, r3_hbm, sent_hbm, recv_hbm,
             acc, idx_v, r0, r1, r2, g0, g1, g2, w0, w1, w2):
        c = lax.axis_index("c")
        ss = lax.axis_index("s")
        slice_base = ss * _NSL
        rows = [r0, r1, r2]
        rsems = [g0, g1, g2]
        ssems = [w0, w1, w2]

        def zrow(i):
            for j in range(8):
                r0[i, pl.ds(j * 16, 16)] = jnp.zeros((16,), jnp.float32)

        pl.loop(0, _CH)(zrow)

        def zcp(i):
            pltpu.sync_copy(r0, acc.at[pl.ds(slice_base + i * _CH, _CH)])

        pl.loop(0, _NSL // _CH)(zcp)

        @pl.when(c == 0)
        def _():
            pltpu.sync_copy(s3_hbm.at[ss], idx_v)

        @pl.when(c == 1)
        def _():
            pltpu.sync_copy(r3_hbm.at[ss], idx_v)

        plsc.subcore_barrier()

        ebase = ss * cpw * _CH

        for b in range(_SNB):
            pltpu.async_copy(e_hbm.at[pl.ds(ebase + b * _CH, _CH)], rows[b],
                             rsems[b])

        def rnd(r):
            for b in range(_SNB):
                pltpu.make_async_copy(e_hbm.at[pl.ds(ebase, _CH)], rows[b],
                                      rsems[b]).wait()
                pltpu.async_copy(rows[b], acc.at[idx_v.at[r * _SNB + b]],
                                 ssems[b], add=True)

            @pl.when(r < snr - 1)
            def _():
                for b in range(_SNB):
                    pltpu.make_async_copy(rows[b], acc.at[idx_v.at[b]],
                                          ssems[b]).wait()
                    k = (r + 1) * _SNB + b
                    pltpu.async_copy(e_hbm.at[pl.ds(ebase + k * _CH, _CH)],
                                     rows[b], rsems[b])

            @pl.when(r == snr - 1)
            def _():
                for b in range(_SNB):
                    pltpu.make_async_copy(rows[b], acc.at[idx_v.at[b]],
                                          ssems[b]).wait()

        pl.loop(0, snr)(rnd)

        for t in range(tail):
            k = snr * _SNB + t
            pltpu.async_copy(e_hbm.at[pl.ds(ebase + k * _CH, _CH)],
                             rows[t], rsems[t])
        for t in range(tail):
            k = snr * _SNB + t
            pltpu.make_async_copy(e_hbm.at[pl.ds(ebase, _CH)], rows[t],
                                  rsems[t]).wait()
            pltpu.async_copy(rows[t], acc.at[idx_v.at[k]], ssems[t],
                             add=True)
        for t in range(tail):
            pltpu.make_async_copy(rows[t], acc.at[idx_v.at[0]],
                                  ssems[t]).wait()

        plsc.subcore_barrier()

        def wcp(i):
            sl = pl.ds(slice_base + i * _CH, _CH)

            @pl.when(c == 0)
            def _():
                pltpu.sync_copy(acc.at[sl], sent_hbm.at[sl])

            @pl.when(c == 1)
            def _():
                pltpu.sync_copy(acc.at[sl], recv_hbm.at[sl])

        pl.loop(0, _NSL // _CH)(wcp)

    f = pl.kernel(
        body,
        out_type=[
            jax.ShapeDtypeStruct((N_PAD, D), jnp.float32),
            jax.ShapeDtypeStruct((N_PAD, D), jnp.float32),
        ],
        mesh=plsc.VectorSubcoreMesh(**_SC_MESH),
        scratch_types=(
            [
                pltpu.VMEM_SHARED((N_PAD, D), jnp.float32),
                pltpu.VMEM((cpw, _CH), jnp.int32),
            ]
            + [pltpu.VMEM((_CH, D), jnp.float32) for _ in range(_SNB)]
            + [pltpu.SemaphoreType.DMA for _ in range(2 * _SNB)]
        ),
    )
    return f(e, s3d, r3d)


def _softplus(x):
    return jnp.log(1.0 + jnp.exp(x))


def _sample_mlp(layers, key):
    ks = jax.random.split(key, len(layers))
    out = []
    for p, k in zip(layers, ks):
        w = p['w_mu'] + jax.random.normal(k, p['w_mu'].shape,
                                          dtype=jnp.float32) * _softplus(p['w_rho'])
        b = p['b_mu'] + jax.random.normal(k, p['b_mu'].shape,
                                          dtype=jnp.float32) * _softplus(p['b_rho'])
        out.append((w, b))
    return out


def kernel(nodes, edges, senders, receivers, globals_, positions, box, params):
    keys = jax.random.split(jax.random.key(42), 4)
    emb = params['embed']

    zero_vec = jnp.zeros((1, D), jnp.float32)
    eye = jnp.eye(D, dtype=jnp.float32)
    h1 = 16 * _H1C * _CH
    s3d1 = senders[:h1].reshape(16, _H1C, _CH)
    r3d1 = receivers[:h1].reshape(16, _H1C, _CH)
    s3d2 = senders[h1:].reshape(16, _H2C, _CH)
    r3d2 = receivers[h1:].reshape(16, _H2C, _CH)

    sw = []
    for s in range(2):
        sp = params['steps'][s]
        k_e, k_n, k_g = jax.random.split(keys[s], 3)
        sw.append((_sample_mlp(sp['edge'], k_e),
                   _sample_mlp(sp['node'], k_n),
                   _sample_mlp(sp['glob'], k_g)))
    (rw1, rb1), (rw2, rb2) = _sample_mlp(params['readout'], keys[-1])

    ew1_0 = sw[0][0][0][0]
    n, ps, pr = _embed_nodes(nodes, emb['node_w'], emb['node_b'],
                             ew1_0[D:2 * D], ew1_0[2 * D:3 * D])
    g = _mm_bias(globals_, emb['glob_w'], emb['glob_b'], blk=1)

    e1 = e2 = None
    for s in range(2):
        (ew1, eb1), (ew2, eb2) = sw[s][0]
        (nw1, nb1), (nw2, nb2) = sw[s][1]
        (gw1, gb1), (gw2, gb2) = sw[s][2]
        w1e = ew1[0:D]
        w1g = ew1[3 * D:4 * D]

        if s == 0:
            x1, x2, xoff2 = edges, edges, h1 // 1280
            wa, wb = emb['edge_w'], w1e
            ev, em = emb['edge_b'].reshape(1, D), w1e
        else:
            x1, x2, xoff2 = e1, e2, 0
            wa, wb = w1e, eye
            ev, em = zero_vec, eye

        gs1, gr1 = _sc_gather(ps, pr, s3d1, r3d1, _H1C)
        gs2, gr2 = _sc_gather(ps, pr, s3d2, r3d2, _H2C)
        e1, ea1 = _edge_mlp(x1, gs1, gr1, wa, wb, ew2, eb2,
                            g, w1g, eb1, ev, em, ne=h1, xoff=0)
        sp1, rp1 = _sc_segment_sums(e1, s3d1, r3d1, _H1C)
        e2, ea2 = _edge_mlp(x2, gs2, gr2, wa, wb, ew2, eb2,
                            g, w1g, eb1, ev, em,
                            ne=N_EDGES - h1, xoff=xoff2)
        sp2, rp2 = _sc_segment_sums(e2, s3d2, r3d2, _H2C)

        if s == 0:
            ew1_n = sw[1][0][0][0]
            n, n_agg, ps, pr = _node_mlp(
                n, sp1, sp2, rp1, rp2,
                nw1[0:D], nw1[D:2 * D], nw1[2 * D:3 * D],
                g, nw1[3 * D:4 * D], nb1, nw2, nb2,
                ws_next=ew1_n[D:2 * D], wr_next=ew1_n[2 * D:3 * D])
        else:
            n, n_agg = _node_mlp(
                n, sp1, sp2, rp1, rp2,
                nw1[0:D], nw1[D:2 * D], nw1[2 * D:3 * D],
                g, nw1[3 * D:4 * D], nb1, nw2, nb2)
        g = _glob_mlp(n_agg, ea1, ea2, g,
                      gw1[0:D], gw1[D:2 * D], gw1[2 * D:3 * D], gb1,
                      gw2, gb2)

    return _readout(g, rw1, rb1, rw2, rb2)
